# baseline (speedup 1.0000x reference)
import jax
import jax.numpy as jnp
from jax.experimental import pallas as pl
from jax.experimental.pallas import tpu as pltpu

_BS = 512


def _body(pe_ref, x_ref, o_ref):
    o_ref[0, :, :] = x_ref[0, :, :] + pe_ref[...]


def kernel(x, abs_pe):
    B, S, D = x.shape
    pe = abs_pe[0, :S, :]
    grid = (S // _BS, B)
    out = pl.pallas_call(
        _body,
        grid=grid,
        in_specs=[
            pl.BlockSpec((_BS, D), lambda s, b: (s, 0)),
            pl.BlockSpec((1, _BS, D), lambda s, b: (b, s, 0)),
        ],
        out_specs=pl.BlockSpec((1, _BS, D), lambda s, b: (b, s, 0)),
        out_shape=jax.ShapeDtypeStruct(x.shape, x.dtype),
        compiler_params=pltpu.CompilerParams(
            dimension_semantics=("arbitrary", "arbitrary"),
        ),
    )(pe, x)
    return out


# speedup vs baseline: 1.2414x; 1.2414x over previous
import jax
import jax.numpy as jnp
from jax.experimental import pallas as pl
from jax.experimental.pallas import tpu as pltpu

_BS = 512


def _body(pe_ref, x_ref, o_ref):
    o_ref[0, :, :] = x_ref[0, :, :] + pe_ref[0, :, :]


def kernel(x, abs_pe):
    B, S, D = x.shape
    grid = (S // _BS, B)
    out = pl.pallas_call(
        _body,
        grid=grid,
        in_specs=[
            pl.BlockSpec((1, _BS, D), lambda s, b: (0, s, 0)),
            pl.BlockSpec((1, _BS, D), lambda s, b: (b, s, 0)),
        ],
        out_specs=pl.BlockSpec((1, _BS, D), lambda s, b: (b, s, 0)),
        out_shape=jax.ShapeDtypeStruct(x.shape, x.dtype),
        compiler_params=pltpu.CompilerParams(
            dimension_semantics=("arbitrary", "arbitrary"),
        ),
    )(abs_pe, x)
    return out
